# baseline (speedup 1.0000x reference)
import functools

import jax
import jax.numpy as jnp
from jax import lax
from jax.experimental import pallas as pl
from jax.experimental.pallas import tpu as pltpu
from jax.experimental.pallas import tpu_sc as plsc

F = 256
N0 = 10000
E = 160000
NC, NS = 2, 16
H0 = N0 // 2

P1, P2, P3 = 5120, 2560, 1280
A1_SZ, A2_SZ, A3_SZ = P1 * P1, P2 * P2, P3 * P3

CH = 80
EPT = E // NS
NCHUNK = EPT // CH
ZCH = 16384

_mesh = plsc.VectorSubcoreMesh(core_axis_name="c", subcore_axis_name="s")


def _zero_region(zv, dst_hbm, base, per_tile, s):
    nch = -(-per_tile // ZCH)
    end = base + NS * per_tile - ZCH
    t0 = base + s * per_tile

    def body(k, _):
        off = jnp.minimum(t0 + k * ZCH, end)
        pltpu.sync_copy(zv, dst_hbm.at[pl.ds(off, ZCH)])
        return _

    lax.fori_loop(0, nch, body, 0)


def _sc_build_body(src_hbm, dst_hbm, zf_hbm,
                   a1_hbm, a2_hbm, a3_hbm,
                   srcv, dstv, i1v, i2v, i3v, onesv, zv, sem):
    c = lax.axis_index("c")
    s = lax.axis_index("s")

    pltpu.sync_copy(zf_hbm, zv)
    _zero_region(zv, a1_hbm, c * (A1_SZ // 2), A1_SZ // (2 * NS), s)
    _zero_region(zv, a2_hbm, c * (A2_SZ // 2), A2_SZ // (2 * NS), s)
    _zero_region(zv, a3_hbm, c * (A3_SZ // 2), A3_SZ // (2 * NS), s)
    for j in range(CH // 16):
        onesv[pl.ds(j * 16, 16)] = jnp.ones((16,), jnp.float32)
    plsc.subcore_barrier()

    ebase = s * EPT
    q1lo = c * (P1 // 2)
    q2lo = c * (P2 // 2)
    q3lo = c * (P3 // 2)

    def chunk(i, _):
        off = ebase + i * CH
        pltpu.sync_copy(src_hbm.at[pl.ds(off, CH)], srcv)
        pltpu.sync_copy(dst_hbm.at[pl.ds(off, CH)], dstv)
        for j in range(CH // 16):
            sl = pl.ds(j * 16, 16)
            sv = srcv[sl]
            dv = dstv[sl]
            p1 = sv >> 1
            q1 = dv >> 1
            p2 = sv >> 2
            q2 = dv >> 2
            p3 = sv >> 3
            q3 = dv >> 3
            ok1 = p1 != q1
            v1 = ok1 & (q1 >= q1lo) & (q1 < q1lo + P1 // 2)
            i1v[sl] = jnp.where(v1, q1 * P1 + p1, A1_SZ + 8 * c)
            ok2 = ok1 & (p2 != q2)
            v2 = ok2 & (q2 >= q2lo) & (q2 < q2lo + P2 // 2)
            i2v[sl] = jnp.where(v2, q2 * P2 + p2, A2_SZ + 8 * c)
            ok3 = ok2 & (p3 != q3)
            v3 = ok3 & (q3 >= q3lo) & (q3 < q3lo + P3 // 2)
            i3v[sl] = jnp.where(v3, q3 * P3 + p3, A3_SZ + 8 * c)
        pltpu.sync_copy(onesv, a1_hbm.at[i1v])
        pltpu.sync_copy(onesv, a2_hbm.at[i2v])
        pltpu.sync_copy(onesv, a3_hbm.at[i3v])
        return _

    lax.fori_loop(0, NCHUNK, chunk, 0)


_sc_build = pl.kernel(
    _sc_build_body,
    out_type=(
        jax.ShapeDtypeStruct((A1_SZ + 16,), jnp.float32),
        jax.ShapeDtypeStruct((A2_SZ + 16,), jnp.float32),
        jax.ShapeDtypeStruct((A3_SZ + 16,), jnp.float32),
    ),
    mesh=_mesh,
    scratch_types=[
        pltpu.VMEM((CH,), jnp.int32),
        pltpu.VMEM((CH,), jnp.int32),
        pltpu.VMEM((CH,), jnp.int32),
        pltpu.VMEM((CH,), jnp.int32),
        pltpu.VMEM((CH,), jnp.int32),
        pltpu.VMEM((CH,), jnp.float32),
        pltpu.VMEM((ZCH,), jnp.float32),
        pltpu.SemaphoreType.DMA,
    ],
)


OWN = 320
NW = NC * NS
NP0 = NW * OWN
ACCR = OWN + 8
CH2 = 1600
NCH2 = E // CH2
FB = 64
STG = 5760
_NLP = pltpu.CompilerParams(needs_layout_passes=False)


def _sc_part_body(src_hbm, dst_hbm, pk_hbm, cnt_hbm,
                  srcv, dstv, stg, cntv, sem):
    c = lax.axis_index("c")
    s = lax.axis_index("s")
    w = c * NS + s
    base = w * OWN

    def grp(j, cur):
        sl = pl.ds(j * 16, 16)
        dv = dstv[sl]
        sv = srcv[sl]
        dloc = dv - base
        m = (dloc >= 0) & (dloc < OWN)
        key = jnp.where(m, 0, 1).astype(jnp.int32)
        packed = sv * 512 + jnp.where(m, dloc, OWN)
        _, pv = plsc.sort_key_val(key, packed)
        stg[pl.ds(cur, 16)] = pv
        return cur + jnp.sum(m.astype(jnp.int32))

    def scan_chunk(i, cur):
        off = i * CH2
        pltpu.sync_copy(src_hbm.at[pl.ds(off, CH2)], srcv)
        pltpu.sync_copy(dst_hbm.at[pl.ds(off, CH2)], dstv)
        return lax.fori_loop(0, CH2 // 16, grp, cur)

    cur = lax.fori_loop(0, NCH2, scan_chunk, 0)

    for t in range(FB // 16):
        stg[pl.ds(cur + t * 16, 16)] = jnp.full((16,), OWN, jnp.int32)
    pltpu.sync_copy(stg, pk_hbm.at[pl.ds(w * STG, STG)])
    cntv[pl.ds(0, 16)] = jnp.zeros((16,), jnp.int32) + cur
    pltpu.sync_copy(cntv, cnt_hbm.at[pl.ds(w * 16, 16)])


_sc_part = pl.kernel(
    _sc_part_body,
    out_type=(
        jax.ShapeDtypeStruct((NW * STG,), jnp.int32),
        jax.ShapeDtypeStruct((NW * 16,), jnp.int32),
    ),
    mesh=_mesh,
    compiler_params=_NLP,
    scratch_types=[
        pltpu.VMEM((CH2,), jnp.int32),
        pltpu.VMEM((CH2,), jnp.int32),
        pltpu.VMEM((STG,), jnp.int32),
        pltpu.VMEM((16,), jnp.int32),
        pltpu.SemaphoreType.DMA,
    ],
)


def _sc_conv_body(x_hbm, pk_hbm, cnt_hbm, z_hbm, out_hbm,
                  stg, gidx, cntv, rowsv, accv, sem):
    c = lax.axis_index("c")
    s = lax.axis_index("s")
    w = c * NS + s
    base = w * OWN

    pltpu.sync_copy(pk_hbm.at[pl.ds(w * STG, STG)], stg)
    pltpu.sync_copy(cnt_hbm.at[pl.ds(w * 16, 16)], cntv)
    for k in range(OWN // 64):
        pltpu.sync_copy(z_hbm, accv.at[pl.ds(k * 64, 64)])
    pltpu.sync_copy(z_hbm.at[pl.ds(0, ACCR - OWN)], accv.at[pl.ds(OWN, ACCR - OWN)])

    cur = cntv[pl.ds(0, 16)][0]
    nf = (cur + FB - 1) // FB

    def flush(f, _):
        def unpack(g, __):
            gidx[pl.ds(g * 16, 16)] = stg[pl.ds(f * FB + g * 16, 16)] >> 9
            return __
        lax.fori_loop(0, FB // 16, unpack, 0)
        pltpu.async_copy(x_hbm.at[gidx], rowsv, sem).wait()

        def per_g(g, __):
            dloc16 = stg[pl.ds(f * FB + g * 16, 16)] & 511
            evec = g * 16 + lax.iota(jnp.int32, 16)

            def per_k(k, ___):
                for u in range(16):
                    cvec = jnp.zeros((16,), jnp.int32) + (k * 16 + u)
                    vals = plsc.load_gather(rowsv, [evec, cvec])
                    plsc.addupdate_scatter(accv, [dloc16, cvec], vals)
                return ___
            return lax.fori_loop(0, F // 16, per_k, __)
        lax.fori_loop(0, FB // 16, per_g, 0)
        return _

    lax.fori_loop(0, nf, flush, 0)
    pltpu.sync_copy(accv.at[pl.ds(0, OWN)], out_hbm.at[pl.ds(base, OWN)])


_sc_conv = pl.kernel(
    _sc_conv_body,
    out_type=jax.ShapeDtypeStruct((NP0, F), jnp.float32),
    mesh=_mesh,
    compiler_params=_NLP,
    scratch_types=[
        pltpu.VMEM((STG,), jnp.int32),
        pltpu.VMEM((FB,), jnp.int32),
        pltpu.VMEM((16,), jnp.int32),
        pltpu.VMEM((FB, F), jnp.float32),
        pltpu.VMEM((ACCR, F), jnp.float32),
        pltpu.SemaphoreType.DMA,
    ],
)


def _t1_body(s1_ref, deg_ref, x_ref, wl_ref, wr_ref, b_ref, o_ref):
    a = s1_ref[...] / jnp.maximum(deg_ref[...], 1.0)
    o = a * wl_ref[...] + x_ref[...] * wr_ref[...] + b_ref[...]
    o_ref[...] = jnp.maximum(o, 0.0)


def _first_conv(s1, deg, x, wl, wr, b):
    R = 1000
    return pl.pallas_call(
        _t1_body,
        grid=(N0 // R,),
        in_specs=[
            pl.BlockSpec((R, 1), lambda g: (g, 0)),
            pl.BlockSpec((R, 1), lambda g: (g, 0)),
            pl.BlockSpec((R, 1), lambda g: (g, 0)),
            pl.BlockSpec((1, F), lambda g: (0, 0)),
            pl.BlockSpec((1, F), lambda g: (0, 0)),
            pl.BlockSpec((1, F), lambda g: (0, 0)),
        ],
        out_specs=pl.BlockSpec((R, F), lambda g: (g, 0)),
        out_shape=jax.ShapeDtypeStruct((N0, F), jnp.float32),
    )(s1, deg, x, wl, wr, b)


def _t2_body(agg_ref, deg_ref, x_ref, wl_ref, wr_ref, b_ref, o_ref):
    a = agg_ref[...] / jnp.maximum(deg_ref[...], 1.0)
    o = (jnp.dot(a, wl_ref[...], preferred_element_type=jnp.float32)
         + jnp.dot(x_ref[...], wr_ref[...], preferred_element_type=jnp.float32)
         + b_ref[...])
    o_ref[...] = jnp.maximum(o, 0.0)


def _sage_mm(agg, deg, x, wl, wr, b):
    R = 1000
    return pl.pallas_call(
        _t2_body,
        grid=(N0 // R,),
        in_specs=[
            pl.BlockSpec((R, F), lambda g: (g, 0)),
            pl.BlockSpec((R, 1), lambda g: (g, 0)),
            pl.BlockSpec((R, F), lambda g: (g, 0)),
            pl.BlockSpec((F, F), lambda g: (0, 0)),
            pl.BlockSpec((F, F), lambda g: (0, 0)),
            pl.BlockSpec((1, F), lambda g: (0, 0)),
        ],
        out_specs=pl.BlockSpec((R, F), lambda g: (g, 0)),
        out_shape=jax.ShapeDtypeStruct((N0, F), jnp.float32),
    )(agg, deg, x, wl, wr, b)


def _t3_body(at_ref, xk_ref, xi_ref, wl_ref, wr_ref, b_ref, o_ref, acc, dacc):
    k = pl.program_id(1)
    nk = pl.num_programs(1)

    @pl.when(k == 0)
    def _():
        acc[...] = jnp.zeros_like(acc)
        dacc[...] = jnp.zeros_like(dacc)

    a = at_ref[...]
    acc[...] += jnp.dot(a, xk_ref[...], preferred_element_type=jnp.float32)
    dacc[...] += jnp.sum(a, axis=1, keepdims=True)

    @pl.when(k == nk - 1)
    def _():
        m = acc[...] / jnp.maximum(dacc[...], 1.0)
        o = (jnp.dot(m, wl_ref[...], preferred_element_type=jnp.float32)
             + jnp.dot(xi_ref[...], wr_ref[...], preferred_element_type=jnp.float32)
             + b_ref[...])
        o_ref[...] = jnp.maximum(o, 0.0)


def _dense_sage(at, x, wl, wr, b, n, r):
    g = n // r
    return pl.pallas_call(
        _t3_body,
        grid=(g, g),
        in_specs=[
            pl.BlockSpec((r, r), lambda i, k: (i, k)),
            pl.BlockSpec((r, F), lambda i, k: (k, 0)),
            pl.BlockSpec((r, F), lambda i, k: (i, 0)),
            pl.BlockSpec((F, F), lambda i, k: (0, 0)),
            pl.BlockSpec((F, F), lambda i, k: (0, 0)),
            pl.BlockSpec((1, F), lambda i, k: (0, 0)),
        ],
        out_specs=pl.BlockSpec((r, F), lambda i, k: (i, 0)),
        out_shape=jax.ShapeDtypeStruct((n, F), jnp.float32),
        scratch_shapes=[
            pltpu.VMEM((r, F), jnp.float32),
            pltpu.VMEM((r, 1), jnp.float32),
        ],
    )(at, x, x, wl, wr, b)


def _t4_body(x_ref, o_ref):
    o_ref[...] = (x_ref[:, 0, :] + x_ref[:, 1, :]) * 0.5


def _pool(x, r):
    n2 = x.shape[0] // 2
    x3 = x.reshape(n2, 2, F)
    return pl.pallas_call(
        _t4_body,
        grid=(n2 // r,),
        in_specs=[pl.BlockSpec((r, 2, F), lambda g: (g, 0, 0))],
        out_specs=pl.BlockSpec((r, F), lambda g: (g, 0)),
        out_shape=jax.ShapeDtypeStruct((n2, F), jnp.float32),
    )(x3)


def _t6_body(xc_ref, xi_ref, o_ref):
    o_ref[...] = (xc_ref[...][:, None, :] + xi_ref[...]) * 0.5


def _unpool(xc, xinfo, r):
    nc = xc.shape[0]
    xi3 = xinfo.reshape(nc, 2, F)
    out = pl.pallas_call(
        _t6_body,
        grid=(nc // r,),
        in_specs=[
            pl.BlockSpec((r, F), lambda g: (g, 0)),
            pl.BlockSpec((r, 2, F), lambda g: (g, 0, 0)),
        ],
        out_specs=pl.BlockSpec((r, 2, F), lambda g: (g, 0, 0)),
        out_shape=jax.ShapeDtypeStruct((nc, 2, F), jnp.float32),
    )(xc, xi3)
    return out.reshape(nc * 2, F)


def _t8_body(x_ref, w1_ref, b1_ref, w2_ref, b2_ref, w3_ref, b3_ref,
             wf_ref, bf_ref, o_ref):
    h = jnp.maximum(jnp.dot(x_ref[...], w1_ref[...],
                            preferred_element_type=jnp.float32) + b1_ref[...], 0.0)
    h = jnp.maximum(jnp.dot(h, w2_ref[...],
                            preferred_element_type=jnp.float32) + b2_ref[...], 0.0)
    h = jnp.maximum(jnp.dot(h, w3_ref[...],
                            preferred_element_type=jnp.float32) + b3_ref[...], 0.0)
    l = jnp.dot(h, wf_ref[...], preferred_element_type=jnp.float32) + bf_ref[...]
    m = jnp.max(l, axis=1, keepdims=True)
    e = jnp.exp(l - m)
    o_ref[...] = e / jnp.sum(e, axis=1, keepdims=True)


def _mlp(x, w1, b1, w2, b2, w3, b3, wf, bf):
    R = 1000
    return pl.pallas_call(
        _t8_body,
        grid=(N0 // R,),
        in_specs=[
            pl.BlockSpec((R, F), lambda g: (g, 0)),
            pl.BlockSpec((F, 256), lambda g: (0, 0)),
            pl.BlockSpec((1, 256), lambda g: (0, 0)),
            pl.BlockSpec((256, 128), lambda g: (0, 0)),
            pl.BlockSpec((1, 128), lambda g: (0, 0)),
            pl.BlockSpec((128, 64), lambda g: (0, 0)),
            pl.BlockSpec((1, 64), lambda g: (0, 0)),
            pl.BlockSpec((64, 2), lambda g: (0, 0)),
            pl.BlockSpec((1, 2), lambda g: (0, 0)),
        ],
        out_specs=pl.BlockSpec((R, 2), lambda g: (g, 0)),
        out_shape=jax.ShapeDtypeStruct((N0, 2), jnp.float32),
    )(x, w1, b1, w2, b2, w3, b3, wf, bf)


def kernel(x, edge_index, batch, W_first_l, W_first_r, b_first,
           W_pre_l, W_pre_r, b_pre, W_post_l, W_post_r, b_post,
           W_coarse_l, W_coarse_r, b_coarse, W1, b1, W2, b2, W3, b3, Wf, bf):
    src = edge_index[0]
    dst = edge_index[1]
    zf = jnp.zeros((ZCH,), jnp.float32)
    z256 = jnp.zeros((64, F), jnp.float32)

    a1f, a2f, a3f = _sc_build(src, dst, zf)
    a1t = a1f[:A1_SZ].reshape(P1, P1)
    a2t = a2f[:A2_SZ].reshape(P2, P2)
    a3t = a3f[:A3_SZ].reshape(P3, P3)
    pk, cnt = _sc_part(src, dst)

    def conv_agg(h):
        return _sc_conv(h, pk, cnt, z256)[:N0]

    xfirst = jnp.concatenate(
        [x, jnp.ones((N0, 1), jnp.float32), jnp.zeros((N0, F - 2), jnp.float32)],
        axis=1)
    aggs0 = conv_agg(xfirst)
    s1 = aggs0[:, 0:1]
    deg0 = aggs0[:, 1:2]

    h = _first_conv(s1, deg0, x, W_first_l.reshape(1, F),
                    W_first_r.reshape(1, F), b_first.reshape(1, F))
    for i in range(2):
        aggs = conv_agg(h)
        h = _sage_mm(aggs, deg0, h, W_pre_l[i], W_pre_r[i],
                     b_pre[i].reshape(1, F))
    x0 = h

    h = _pool(h, 1000)
    h = jnp.pad(h, ((0, P1 - H0), (0, 0)))
    for i in range(2):
        h = _dense_sage(a1t, h, W_pre_l[i], W_pre_r[i],
                        b_pre[i].reshape(1, F), P1, 1024)
    x1 = h

    h = _pool(h, 512)
    for i in range(2):
        h = _dense_sage(a2t, h, W_pre_l[i], W_pre_r[i],
                        b_pre[i].reshape(1, F), P2, 512)
    x2f = h

    h = _pool(h, 256)
    h = _dense_sage(a3t, h, W_coarse_l, W_coarse_r,
                    b_coarse.reshape(1, F), P3, 256)

    h = _unpool(h, x2f, 256)
    for i in range(2):
        h = _dense_sage(a2t, h, W_post_l[i], W_post_r[i],
                        b_post[i].reshape(1, F), P2, 512)

    h = _unpool(h, x1, 512)
    for i in range(2):
        h = _dense_sage(a1t, h, W_post_l[i], W_post_r[i],
                        b_post[i].reshape(1, F), P1, 1024)

    h = _unpool(h[:H0], x0, 1000)
    for i in range(2):
        aggs = conv_agg(h)
        h = _sage_mm(aggs, deg0, h, W_post_l[i], W_post_r[i],
                     b_post[i].reshape(1, F))

    return _mlp(h, W1, b1.reshape(1, 256), W2, b2.reshape(1, 128),
                W3, b3.reshape(1, 64), Wf, bf.reshape(1, 2))


# speedup vs baseline: 1.0197x; 1.0197x over previous
import functools

import jax
import jax.numpy as jnp
from jax import lax
from jax.experimental import pallas as pl
from jax.experimental.pallas import tpu as pltpu
from jax.experimental.pallas import tpu_sc as plsc

F = 256
N0 = 10000
E = 160000
NC, NS = 2, 16
H0 = N0 // 2

P1, P2, P3 = 5120, 2560, 1280
A1_SZ, A2_SZ, A3_SZ = P1 * P1, P2 * P2, P3 * P3

CH = 80
EPT = E // NS
NCHUNK = EPT // CH
ZCH = 16384

_mesh = plsc.VectorSubcoreMesh(core_axis_name="c", subcore_axis_name="s")


BCH = 2000
NBC = EPT // BCH


def _zero_region_async(zv, dst_hbm, base, per_tile, s, sem):
    nch = -(-per_tile // ZCH)
    end = base + NS * per_tile - ZCH
    t0 = base + s * per_tile

    def body(k, _):
        off = jnp.minimum(t0 + k * ZCH, end)
        pltpu.async_copy(zv, dst_hbm.at[pl.ds(off, ZCH)], sem)
        return _

    lax.fori_loop(0, nch, body, 0)
    return nch


def _drain(zf_hbm, buf, sem, n):
    def body(k, _):
        pltpu.make_async_copy(zf_hbm.at[pl.ds(0, buf.shape[0])], buf, sem).wait()
        return _
    lax.fori_loop(0, n, body, 0)


def _sc_build_body(src_hbm, dst_hbm, zf_hbm,
                   a1_hbm, a2_hbm, a3_hbm,
                   srcv, dstv, i1a, i2a, i3a, onesv, zv, ddum, sem, zsem):
    c = lax.axis_index("c")
    s = lax.axis_index("s")

    pltpu.sync_copy(zf_hbm, zv)
    n1 = _zero_region_async(zv, a1_hbm, c * (A1_SZ // 2), A1_SZ // (2 * NS), s, zsem)
    n2 = _zero_region_async(zv, a2_hbm, c * (A2_SZ // 2), A2_SZ // (2 * NS), s, zsem)
    n3 = _zero_region_async(zv, a3_hbm, c * (A3_SZ // 2), A3_SZ // (2 * NS), s, zsem)
    for j in range(CH // 16):
        onesv[pl.ds(j * 16, 16)] = jnp.ones((16,), jnp.float32)

    ebase = s * EPT
    q1lo = c * (P1 // 2)
    q2lo = c * (P2 // 2)
    q3lo = c * (P3 // 2)

    for b in range(NBC):
        pltpu.sync_copy(src_hbm.at[pl.ds(ebase + b * BCH, BCH)], srcv)
        pltpu.sync_copy(dst_hbm.at[pl.ds(ebase + b * BCH, BCH)], dstv)

        def jgrp(j, _, b=b):
            sl = pl.ds(j * 16, 16)
            row = b * (BCH // CH) + j // (CH // 16)
            csl = pl.ds((j % (CH // 16)) * 16, 16)
            sv = srcv[sl]
            dv = dstv[sl]
            p1 = sv >> 1
            q1 = dv >> 1
            p2 = sv >> 2
            q2 = dv >> 2
            p3 = sv >> 3
            q3 = dv >> 3
            ok1 = p1 != q1
            v1 = ok1 & (q1 >= q1lo) & (q1 < q1lo + P1 // 2)
            i1a[row, csl] = jnp.where(v1, q1 * P1 + p1, A1_SZ + 8 * c)
            ok2 = ok1 & (p2 != q2)
            v2 = ok2 & (q2 >= q2lo) & (q2 < q2lo + P2 // 2)
            i2a[row, csl] = jnp.where(v2, q2 * P2 + p2, A2_SZ + 8 * c)
            ok3 = ok2 & (p3 != q3)
            v3 = ok3 & (q3 >= q3lo) & (q3 < q3lo + P3 // 2)
            i3a[row, csl] = jnp.where(v3, q3 * P3 + p3, A3_SZ + 8 * c)
            return _

        lax.fori_loop(0, BCH // 16, jgrp, 0)

    _drain(zf_hbm, zv, zsem, n1 + n2 + n3)
    plsc.subcore_barrier()

    def fire(g, _):
        pltpu.async_copy(onesv, a1_hbm.at[i1a.at[g]], sem)
        pltpu.async_copy(onesv, a2_hbm.at[i2a.at[g]], sem)
        pltpu.async_copy(onesv, a3_hbm.at[i3a.at[g]], sem)
        return _

    lax.fori_loop(0, EPT // CH, fire, 0)
    _drain(zf_hbm, ddum, sem, (3 * EPT * 4) // (4 * 15000))


_sc_build = pl.kernel(
    _sc_build_body,
    out_type=(
        jax.ShapeDtypeStruct((A1_SZ + 16,), jnp.float32),
        jax.ShapeDtypeStruct((A2_SZ + 16,), jnp.float32),
        jax.ShapeDtypeStruct((A3_SZ + 16,), jnp.float32),
    ),
    mesh=_mesh,
    scratch_types=[
        pltpu.VMEM((BCH,), jnp.int32),
        pltpu.VMEM((BCH,), jnp.int32),
        pltpu.VMEM((EPT // CH, CH), jnp.int32),
        pltpu.VMEM((EPT // CH, CH), jnp.int32),
        pltpu.VMEM((EPT // CH, CH), jnp.int32),
        pltpu.VMEM((CH,), jnp.float32),
        pltpu.VMEM((ZCH,), jnp.float32),
        pltpu.VMEM((15000,), jnp.float32),
        pltpu.SemaphoreType.DMA,
        pltpu.SemaphoreType.DMA,
    ],
)


OWN = 320
NW = NC * NS
NP0 = NW * OWN
ACCR = OWN + 8
CH2 = 4000
NCH2 = E // CH2
FB = 64
STG = 5760
_NLP = pltpu.CompilerParams(needs_layout_passes=False)


def _sc_part_body(src_hbm, dst_hbm, pk_hbm, cnt_hbm,
                  s0, d0, s1, d1, stg, cntv, sem0, sem1):
    c = lax.axis_index("c")
    s = lax.axis_index("s")
    w = c * NS + s
    base = w * OWN

    def fire(i, sv, dv, sem):
        pltpu.async_copy(src_hbm.at[pl.ds(i * CH2, CH2)], sv, sem)
        pltpu.async_copy(dst_hbm.at[pl.ds(i * CH2, CH2)], dv, sem)

    def wait2(sv, sem):
        pltpu.make_async_copy(src_hbm.at[pl.ds(0, CH2)], sv, sem).wait()
        pltpu.make_async_copy(src_hbm.at[pl.ds(0, CH2)], sv, sem).wait()

    def process(srcv, dstv, cur):
        def grp(j, cur):
            sl = pl.ds(j * 16, 16)
            dv = dstv[sl]
            sv = srcv[sl]
            dloc = dv - base
            m = (dloc >= 0) & (dloc < OWN)
            key = jnp.where(m, 0, 1).astype(jnp.int32)
            packed = sv * 512 + jnp.where(m, dloc, OWN)
            _, pv = plsc.sort_key_val(key, packed)
            stg[pl.ds(cur, 16)] = pv
            return cur + jnp.sum(m.astype(jnp.int32))
        return lax.fori_loop(0, CH2 // 16, grp, cur)

    fire(0, s0, d0, sem0)
    fire(1, s1, d1, sem1)

    def pair(fp, cur):
        i0 = 2 * fp
        wait2(s0, sem0)
        cur = process(s0, d0, cur)

        @pl.when(i0 + 2 < NCH2)
        def _f0():
            fire(i0 + 2, s0, d0, sem0)
        wait2(s1, sem1)
        cur = process(s1, d1, cur)

        @pl.when(i0 + 3 < NCH2)
        def _f1():
            fire(i0 + 3, s1, d1, sem1)
        return cur

    cur = lax.fori_loop(0, NCH2 // 2, pair, 0)

    for t in range(FB // 16):
        stg[pl.ds(cur + t * 16, 16)] = jnp.full((16,), OWN, jnp.int32)
    pltpu.sync_copy(stg, pk_hbm.at[pl.ds(w * STG, STG)])
    cntv[pl.ds(0, 16)] = jnp.zeros((16,), jnp.int32) + cur
    pltpu.sync_copy(cntv, cnt_hbm.at[pl.ds(w * 16, 16)])


_sc_part = pl.kernel(
    _sc_part_body,
    out_type=(
        jax.ShapeDtypeStruct((NW * STG,), jnp.int32),
        jax.ShapeDtypeStruct((NW * 16,), jnp.int32),
    ),
    mesh=_mesh,
    compiler_params=_NLP,
    scratch_types=[
        pltpu.VMEM((CH2,), jnp.int32),
        pltpu.VMEM((CH2,), jnp.int32),
        pltpu.VMEM((CH2,), jnp.int32),
        pltpu.VMEM((CH2,), jnp.int32),
        pltpu.VMEM((STG,), jnp.int32),
        pltpu.VMEM((16,), jnp.int32),
        pltpu.SemaphoreType.DMA,
        pltpu.SemaphoreType.DMA,
    ],
)


def _sc_conv_body(x_hbm, pk_hbm, cnt_hbm, zacc_hbm, out_hbm,
                  stg, gidx, cntv, rows0, rows1, accv, sem0, sem1):
    c = lax.axis_index("c")
    s = lax.axis_index("s")
    w = c * NS + s
    base = w * OWN

    pltpu.sync_copy(pk_hbm.at[pl.ds(w * STG, STG)], stg)
    pltpu.sync_copy(cnt_hbm.at[pl.ds(w * 16, 16)], cntv)
    pltpu.sync_copy(zacc_hbm, accv)

    cur = cntv[pl.ds(0, 16)][0]
    nf = (cur + FB - 1) // FB

    def unpack(g, _):
        gidx[pl.ds(g * 16, 16)] = stg[pl.ds(g * 16, 16)] >> 9
        return _
    lax.fori_loop(0, STG // 16, unpack, 0)

    def fire(f, buf, sem):
        pltpu.async_copy(x_hbm.at[gidx.at[pl.ds(f * FB, FB)]], buf, sem)

    def waitg(buf, sem):
        pltpu.make_async_copy(x_hbm.at[pl.ds(0, FB)], buf, sem).wait()

    def accum(f, buf):
        def per_g(g, __):
            dloc16 = stg[pl.ds(f * FB + g * 16, 16)] & 511
            evec = g * 16 + lax.iota(jnp.int32, 16)

            def per_k(k, ___):
                for u in range(16):
                    cvec = jnp.zeros((16,), jnp.int32) + (k * 16 + u)
                    vals = plsc.load_gather(buf, [evec, cvec])
                    plsc.addupdate_scatter(accv, [dloc16, cvec], vals)
                return ___
            return lax.fori_loop(0, F // 16, per_k, __)
        lax.fori_loop(0, FB // 16, per_g, 0)

    @pl.when(nf > 0)
    def _():
        fire(0, rows0, sem0)

    @pl.when(nf > 1)
    def _():
        fire(1, rows1, sem1)

    def pair(fp, carry):
        f0 = 2 * fp
        f1 = f0 + 1

        @pl.when(f0 < nf)
        def _b0():
            waitg(rows0, sem0)
            accum(f0, rows0)

            @pl.when(f0 + 2 < nf)
            def _b0f():
                fire(f0 + 2, rows0, sem0)

        @pl.when(f1 < nf)
        def _b1():
            waitg(rows1, sem1)
            accum(f1, rows1)

            @pl.when(f1 + 2 < nf)
            def _b1f():
                fire(f1 + 2, rows1, sem1)
        return carry

    lax.fori_loop(0, (STG // FB + 1) // 2, pair, 0)
    pltpu.sync_copy(accv.at[pl.ds(0, OWN)], out_hbm.at[pl.ds(base, OWN)])


_sc_conv = pl.kernel(
    _sc_conv_body,
    out_type=jax.ShapeDtypeStruct((NP0, F), jnp.float32),
    mesh=_mesh,
    compiler_params=_NLP,
    scratch_types=[
        pltpu.VMEM((STG,), jnp.int32),
        pltpu.VMEM((STG,), jnp.int32),
        pltpu.VMEM((16,), jnp.int32),
        pltpu.VMEM((FB, F), jnp.float32),
        pltpu.VMEM((FB, F), jnp.float32),
        pltpu.VMEM((ACCR, F), jnp.float32),
        pltpu.SemaphoreType.DMA,
        pltpu.SemaphoreType.DMA,
    ],
)


def _t1_body(s1_ref, deg_ref, x_ref, wl_ref, wr_ref, b_ref, o_ref):
    a = s1_ref[...] / jnp.maximum(deg_ref[...], 1.0)
    o = a * wl_ref[...] + x_ref[...] * wr_ref[...] + b_ref[...]
    o_ref[...] = jnp.maximum(o, 0.0)


def _first_conv(s1, deg, x, wl, wr, b):
    R = 1000
    return pl.pallas_call(
        _t1_body,
        grid=(N0 // R,),
        in_specs=[
            pl.BlockSpec((R, 1), lambda g: (g, 0)),
            pl.BlockSpec((R, 1), lambda g: (g, 0)),
            pl.BlockSpec((R, 1), lambda g: (g, 0)),
            pl.BlockSpec((1, F), lambda g: (0, 0)),
            pl.BlockSpec((1, F), lambda g: (0, 0)),
            pl.BlockSpec((1, F), lambda g: (0, 0)),
        ],
        out_specs=pl.BlockSpec((R, F), lambda g: (g, 0)),
        out_shape=jax.ShapeDtypeStruct((N0, F), jnp.float32),
    )(s1, deg, x, wl, wr, b)


def _t2_body(agg_ref, deg_ref, x_ref, wl_ref, wr_ref, b_ref, o_ref):
    a = agg_ref[...] / jnp.maximum(deg_ref[...], 1.0)
    o = (jnp.dot(a, wl_ref[...], preferred_element_type=jnp.float32)
         + jnp.dot(x_ref[...], wr_ref[...], preferred_element_type=jnp.float32)
         + b_ref[...])
    o_ref[...] = jnp.maximum(o, 0.0)


def _sage_mm(agg, deg, x, wl, wr, b):
    R = 1000
    return pl.pallas_call(
        _t2_body,
        grid=(N0 // R,),
        in_specs=[
            pl.BlockSpec((R, F), lambda g: (g, 0)),
            pl.BlockSpec((R, 1), lambda g: (g, 0)),
            pl.BlockSpec((R, F), lambda g: (g, 0)),
            pl.BlockSpec((F, F), lambda g: (0, 0)),
            pl.BlockSpec((F, F), lambda g: (0, 0)),
            pl.BlockSpec((1, F), lambda g: (0, 0)),
        ],
        out_specs=pl.BlockSpec((R, F), lambda g: (g, 0)),
        out_shape=jax.ShapeDtypeStruct((N0, F), jnp.float32),
    )(agg, deg, x, wl, wr, b)


def _t3_body(at_ref, xk_ref, xi_ref, wl_ref, wr_ref, b_ref, o_ref, acc, dacc):
    k = pl.program_id(1)
    nk = pl.num_programs(1)

    @pl.when(k == 0)
    def _():
        acc[...] = jnp.zeros_like(acc)
        dacc[...] = jnp.zeros_like(dacc)

    a = at_ref[...]
    acc[...] += jnp.dot(a, xk_ref[...], preferred_element_type=jnp.float32)
    dacc[...] += jnp.sum(a, axis=1, keepdims=True)

    @pl.when(k == nk - 1)
    def _():
        m = acc[...] / jnp.maximum(dacc[...], 1.0)
        o = (jnp.dot(m, wl_ref[...], preferred_element_type=jnp.float32)
             + jnp.dot(xi_ref[...], wr_ref[...], preferred_element_type=jnp.float32)
             + b_ref[...])
        o_ref[...] = jnp.maximum(o, 0.0)


def _dense_sage(at, x, wl, wr, b, n, r):
    g = n // r
    return pl.pallas_call(
        _t3_body,
        grid=(g, g),
        in_specs=[
            pl.BlockSpec((r, r), lambda i, k: (i, k)),
            pl.BlockSpec((r, F), lambda i, k: (k, 0)),
            pl.BlockSpec((r, F), lambda i, k: (i, 0)),
            pl.BlockSpec((F, F), lambda i, k: (0, 0)),
            pl.BlockSpec((F, F), lambda i, k: (0, 0)),
            pl.BlockSpec((1, F), lambda i, k: (0, 0)),
        ],
        out_specs=pl.BlockSpec((r, F), lambda i, k: (i, 0)),
        out_shape=jax.ShapeDtypeStruct((n, F), jnp.float32),
        scratch_shapes=[
            pltpu.VMEM((r, F), jnp.float32),
            pltpu.VMEM((r, 1), jnp.float32),
        ],
    )(at, x, x, wl, wr, b)


def _t4_body(x_ref, o_ref):
    o_ref[...] = (x_ref[:, 0, :] + x_ref[:, 1, :]) * 0.5


def _pool(x, r):
    n2 = x.shape[0] // 2
    x3 = x.reshape(n2, 2, F)
    return pl.pallas_call(
        _t4_body,
        grid=(n2 // r,),
        in_specs=[pl.BlockSpec((r, 2, F), lambda g: (g, 0, 0))],
        out_specs=pl.BlockSpec((r, F), lambda g: (g, 0)),
        out_shape=jax.ShapeDtypeStruct((n2, F), jnp.float32),
    )(x3)


def _t6_body(xc_ref, xi_ref, o_ref):
    o_ref[...] = (xc_ref[...][:, None, :] + xi_ref[...]) * 0.5


def _unpool(xc, xinfo, r):
    nc = xc.shape[0]
    xi3 = xinfo.reshape(nc, 2, F)
    out = pl.pallas_call(
        _t6_body,
        grid=(nc // r,),
        in_specs=[
            pl.BlockSpec((r, F), lambda g: (g, 0)),
            pl.BlockSpec((r, 2, F), lambda g: (g, 0, 0)),
        ],
        out_specs=pl.BlockSpec((r, 2, F), lambda g: (g, 0, 0)),
        out_shape=jax.ShapeDtypeStruct((nc, 2, F), jnp.float32),
    )(xc, xi3)
    return out.reshape(nc * 2, F)


def _t8_body(x_ref, w1_ref, b1_ref, w2_ref, b2_ref, w3_ref, b3_ref,
             wf_ref, bf_ref, o_ref):
    h = jnp.maximum(jnp.dot(x_ref[...], w1_ref[...],
                            preferred_element_type=jnp.float32) + b1_ref[...], 0.0)
    h = jnp.maximum(jnp.dot(h, w2_ref[...],
                            preferred_element_type=jnp.float32) + b2_ref[...], 0.0)
    h = jnp.maximum(jnp.dot(h, w3_ref[...],
                            preferred_element_type=jnp.float32) + b3_ref[...], 0.0)
    l = jnp.dot(h, wf_ref[...], preferred_element_type=jnp.float32) + bf_ref[...]
    m = jnp.max(l, axis=1, keepdims=True)
    e = jnp.exp(l - m)
    o_ref[...] = e / jnp.sum(e, axis=1, keepdims=True)


def _mlp(x, w1, b1, w2, b2, w3, b3, wf, bf):
    R = 1000
    return pl.pallas_call(
        _t8_body,
        grid=(N0 // R,),
        in_specs=[
            pl.BlockSpec((R, F), lambda g: (g, 0)),
            pl.BlockSpec((F, 256), lambda g: (0, 0)),
            pl.BlockSpec((1, 256), lambda g: (0, 0)),
            pl.BlockSpec((256, 128), lambda g: (0, 0)),
            pl.BlockSpec((1, 128), lambda g: (0, 0)),
            pl.BlockSpec((128, 64), lambda g: (0, 0)),
            pl.BlockSpec((1, 64), lambda g: (0, 0)),
            pl.BlockSpec((64, 2), lambda g: (0, 0)),
            pl.BlockSpec((1, 2), lambda g: (0, 0)),
        ],
        out_specs=pl.BlockSpec((R, 2), lambda g: (g, 0)),
        out_shape=jax.ShapeDtypeStruct((N0, 2), jnp.float32),
    )(x, w1, b1, w2, b2, w3, b3, wf, bf)


def kernel(x, edge_index, batch, W_first_l, W_first_r, b_first,
           W_pre_l, W_pre_r, b_pre, W_post_l, W_post_r, b_post,
           W_coarse_l, W_coarse_r, b_coarse, W1, b1, W2, b2, W3, b3, Wf, bf):
    src = edge_index[0]
    dst = edge_index[1]
    zf = jnp.zeros((ZCH,), jnp.float32)
    zacc = jnp.zeros((ACCR, F), jnp.float32)

    a1f, a2f, a3f = _sc_build(src, dst, zf)
    a1t = a1f[:A1_SZ].reshape(P1, P1)
    a2t = a2f[:A2_SZ].reshape(P2, P2)
    a3t = a3f[:A3_SZ].reshape(P3, P3)
    pk, cnt = _sc_part(src, dst)

    def conv_agg(h):
        return _sc_conv(h, pk, cnt, zacc)[:N0]

    xfirst = jnp.concatenate(
        [x, jnp.ones((N0, 1), jnp.float32), jnp.zeros((N0, F - 2), jnp.float32)],
        axis=1)
    aggs0 = conv_agg(xfirst)
    s1 = aggs0[:, 0:1]
    deg0 = aggs0[:, 1:2]

    h = _first_conv(s1, deg0, x, W_first_l.reshape(1, F),
                    W_first_r.reshape(1, F), b_first.reshape(1, F))
    for i in range(2):
        aggs = conv_agg(h)
        h = _sage_mm(aggs, deg0, h, W_pre_l[i], W_pre_r[i],
                     b_pre[i].reshape(1, F))
    x0 = h

    h = _pool(h, 1000)
    h = jnp.pad(h, ((0, P1 - H0), (0, 0)))
    for i in range(2):
        h = _dense_sage(a1t, h, W_pre_l[i], W_pre_r[i],
                        b_pre[i].reshape(1, F), P1, 1024)
    x1 = h

    h = _pool(h, 512)
    for i in range(2):
        h = _dense_sage(a2t, h, W_pre_l[i], W_pre_r[i],
                        b_pre[i].reshape(1, F), P2, 512)
    x2f = h

    h = _pool(h, 256)
    h = _dense_sage(a3t, h, W_coarse_l, W_coarse_r,
                    b_coarse.reshape(1, F), P3, 256)

    h = _unpool(h, x2f, 256)
    for i in range(2):
        h = _dense_sage(a2t, h, W_post_l[i], W_post_r[i],
                        b_post[i].reshape(1, F), P2, 512)

    h = _unpool(h, x1, 512)
    for i in range(2):
        h = _dense_sage(a1t, h, W_post_l[i], W_post_r[i],
                        b_post[i].reshape(1, F), P1, 1024)

    h = _unpool(h[:H0], x0, 1000)
    for i in range(2):
        aggs = conv_agg(h)
        h = _sage_mm(aggs, deg0, h, W_post_l[i], W_post_r[i],
                     b_post[i].reshape(1, F))

    return _mlp(h, W1, b1.reshape(1, 256), W2, b2.reshape(1, 128),
                W3, b3.reshape(1, 64), Wf, bf.reshape(1, 2))


# speedup vs baseline: 3.4964x; 3.4289x over previous
import functools

import jax
import jax.numpy as jnp
from jax import lax
from jax.experimental import pallas as pl
from jax.experimental.pallas import tpu as pltpu
from jax.experimental.pallas import tpu_sc as plsc

F = 256
N0 = 10000
E = 160000
NC, NS = 2, 16
H0 = N0 // 2

P1, P2, P3 = 5120, 2560, 1280
A1_SZ, A2_SZ, A3_SZ = P1 * P1, P2 * P2, P3 * P3

CH = 80
EPT = E // NS
NCHUNK = EPT // CH
ZCH = 16384

_mesh = plsc.VectorSubcoreMesh(core_axis_name="c", subcore_axis_name="s")


OWN = 320
NW = NC * NS
NP0 = NW * OWN
ACCR = OWN + 8
CH2 = 4000
NCH2 = E // CH2
FB = 64
STG = 5760
_NLP = pltpu.CompilerParams(needs_layout_passes=False)


def _sc_part_body(src_hbm, dst_hbm, pk_hbm, cnt_hbm,
                  s0, d0, s1, d1, stg, cntv, sem0, sem1):
    c = lax.axis_index("c")
    s = lax.axis_index("s")
    w = c * NS + s
    base = w * OWN

    def fire(i, sv, dv, sem):
        pltpu.async_copy(src_hbm.at[pl.ds(i * CH2, CH2)], sv, sem)
        pltpu.async_copy(dst_hbm.at[pl.ds(i * CH2, CH2)], dv, sem)

    def wait2(sv, sem):
        pltpu.make_async_copy(src_hbm.at[pl.ds(0, CH2)], sv, sem).wait()
        pltpu.make_async_copy(src_hbm.at[pl.ds(0, CH2)], sv, sem).wait()

    def process(srcv, dstv, cur):
        def grp(j, cur):
            sl = pl.ds(j * 16, 16)
            dv = dstv[sl]
            sv = srcv[sl]
            dloc = dv - base
            m = (dloc >= 0) & (dloc < OWN)
            key = jnp.where(m, 0, 1).astype(jnp.int32)
            packed = sv * 512 + jnp.where(m, dloc, OWN)
            _, pv = plsc.sort_key_val(key, packed)
            stg[pl.ds(cur, 16)] = pv
            return cur + jnp.sum(m.astype(jnp.int32))
        return lax.fori_loop(0, CH2 // 16, grp, cur)

    fire(0, s0, d0, sem0)
    fire(1, s1, d1, sem1)

    def pair(fp, cur):
        i0 = 2 * fp
        wait2(s0, sem0)
        cur = process(s0, d0, cur)

        @pl.when(i0 + 2 < NCH2)
        def _f0():
            fire(i0 + 2, s0, d0, sem0)
        wait2(s1, sem1)
        cur = process(s1, d1, cur)

        @pl.when(i0 + 3 < NCH2)
        def _f1():
            fire(i0 + 3, s1, d1, sem1)
        return cur

    cur = lax.fori_loop(0, NCH2 // 2, pair, 0)

    for t in range(FB // 16):
        stg[pl.ds(cur + t * 16, 16)] = jnp.full((16,), OWN, jnp.int32)
    pltpu.sync_copy(stg, pk_hbm.at[pl.ds(w * STG, STG)])
    cntv[pl.ds(0, 16)] = jnp.zeros((16,), jnp.int32) + cur
    pltpu.sync_copy(cntv, cnt_hbm.at[pl.ds(w * 16, 16)])


_sc_part = pl.kernel(
    _sc_part_body,
    out_type=(
        jax.ShapeDtypeStruct((NW * STG,), jnp.int32),
        jax.ShapeDtypeStruct((NW * 16,), jnp.int32),
    ),
    mesh=_mesh,
    compiler_params=_NLP,
    scratch_types=[
        pltpu.VMEM((CH2,), jnp.int32),
        pltpu.VMEM((CH2,), jnp.int32),
        pltpu.VMEM((CH2,), jnp.int32),
        pltpu.VMEM((CH2,), jnp.int32),
        pltpu.VMEM((STG,), jnp.int32),
        pltpu.VMEM((16,), jnp.int32),
        pltpu.SemaphoreType.DMA,
        pltpu.SemaphoreType.DMA,
    ],
)


RBW = 40960
BTRASH = RBW


def _sc_build_body(pk_hbm, cnt_hbm, zr_hbm, a1_hbm, a2_hbm, a3_hbm,
                   stg, cntv, rowb, sem):
    c = lax.axis_index("c")
    s = lax.axis_index("s")
    w = c * NS + s
    pltpu.sync_copy(pk_hbm.at[pl.ds(w * STG, STG)], stg)
    pltpu.sync_copy(cnt_hbm.at[pl.ds(w * 16, 16)], cntv)
    cur = cntv[pl.ds(0, 16)][0]
    iota16 = lax.iota(jnp.int32, 16)
    ones16 = jnp.ones((16,), jnp.float32)

    def slab(a_hbm, shift, cols, rpc, nchunk):
        cw = rpc * cols
        sw = nchunk * cw

        def chunk(ch, carry):
            pltpu.sync_copy(zr_hbm.at[pl.ds(0, cw)], rowb.at[pl.ds(0, cw)])

            def grp(j, inner):
                pk16 = stg[pl.ds(j * 16, 16)]
                valid = (j * 16 + iota16) < cur
                sv = pk16 >> 9
                dloc = pk16 & 511
                ok = (sv >> 1) != ((dloc >> 1) + 160 * w)
                if shift >= 2:
                    ok = ok & ((sv >> 2) != ((dloc >> 2) + 80 * w))
                if shift >= 3:
                    ok = ok & ((sv >> 3) != ((dloc >> 3) + 40 * w))
                rl = (dloc >> shift) - ch * rpc
                p = sv >> shift
                m = valid & ok & (rl >= 0) & (rl < rpc)
                idx = jnp.where(m, rl * cols + p, BTRASH)
                plsc.addupdate_scatter(rowb, [idx], ones16)
                return inner

            lax.fori_loop(0, STG // 16, grp, 0)
            pltpu.sync_copy(rowb.at[pl.ds(0, cw)],
                            a_hbm.at[pl.ds(w * sw + ch * cw, cw)])
            return carry

        lax.fori_loop(0, nchunk, chunk, 0)

    slab(a1_hbm, 1, P1, 8, 20)
    slab(a2_hbm, 2, P2, 16, 5)
    slab(a3_hbm, 3, P3, 20, 2)


_sc_build = pl.kernel(
    _sc_build_body,
    out_type=(
        jax.ShapeDtypeStruct((A1_SZ,), jnp.float32),
        jax.ShapeDtypeStruct((A2_SZ,), jnp.float32),
        jax.ShapeDtypeStruct((A3_SZ,), jnp.float32),
    ),
    mesh=_mesh,
    compiler_params=_NLP,
    scratch_types=[
        pltpu.VMEM((STG,), jnp.int32),
        pltpu.VMEM((16,), jnp.int32),
        pltpu.VMEM((RBW + 16,), jnp.float32),
        pltpu.SemaphoreType.DMA,
    ],
)


def _sc_conv_body(x_hbm, pk_hbm, cnt_hbm, zacc_hbm, out_hbm,
                  stg, gidx, cntv, rows0, rows1, accv, sem0, sem1):
    c = lax.axis_index("c")
    s = lax.axis_index("s")
    w = c * NS + s
    base = w * OWN

    pltpu.sync_copy(pk_hbm.at[pl.ds(w * STG, STG)], stg)
    pltpu.sync_copy(cnt_hbm.at[pl.ds(w * 16, 16)], cntv)
    pltpu.sync_copy(zacc_hbm, accv)

    cur = cntv[pl.ds(0, 16)][0]
    nf = (cur + FB - 1) // FB

    def unpack(g, _):
        gidx[pl.ds(g * 16, 16)] = stg[pl.ds(g * 16, 16)] >> 9
        return _
    lax.fori_loop(0, STG // 16, unpack, 0)

    def fire(f, buf, sem):
        pltpu.async_copy(x_hbm.at[gidx.at[pl.ds(f * FB, FB)]], buf, sem)

    def waitg(buf, sem):
        pltpu.make_async_copy(x_hbm.at[pl.ds(0, FB)], buf, sem).wait()

    def accum(f, buf):
        def per_g(g, __):
            dloc16 = stg[pl.ds(f * FB + g * 16, 16)] & 511
            evec = g * 16 + lax.iota(jnp.int32, 16)

            def per_k(k, ___):
                for u in range(16):
                    cvec = jnp.zeros((16,), jnp.int32) + (k * 16 + u)
                    vals = plsc.load_gather(buf, [evec, cvec])
                    plsc.addupdate_scatter(accv, [dloc16, cvec], vals)
                return ___
            return lax.fori_loop(0, F // 16, per_k, __)
        lax.fori_loop(0, FB // 16, per_g, 0)

    @pl.when(nf > 0)
    def _():
        fire(0, rows0, sem0)

    @pl.when(nf > 1)
    def _():
        fire(1, rows1, sem1)

    def pair(fp, carry):
        f0 = 2 * fp
        f1 = f0 + 1

        @pl.when(f0 < nf)
        def _b0():
            waitg(rows0, sem0)
            accum(f0, rows0)

            @pl.when(f0 + 2 < nf)
            def _b0f():
                fire(f0 + 2, rows0, sem0)

        @pl.when(f1 < nf)
        def _b1():
            waitg(rows1, sem1)
            accum(f1, rows1)

            @pl.when(f1 + 2 < nf)
            def _b1f():
                fire(f1 + 2, rows1, sem1)
        return carry

    lax.fori_loop(0, (STG // FB + 1) // 2, pair, 0)
    pltpu.sync_copy(accv.at[pl.ds(0, OWN)], out_hbm.at[pl.ds(base, OWN)])


_sc_conv = pl.kernel(
    _sc_conv_body,
    out_type=jax.ShapeDtypeStruct((NP0, F), jnp.float32),
    mesh=_mesh,
    compiler_params=_NLP,
    scratch_types=[
        pltpu.VMEM((STG,), jnp.int32),
        pltpu.VMEM((STG,), jnp.int32),
        pltpu.VMEM((16,), jnp.int32),
        pltpu.VMEM((FB, F), jnp.float32),
        pltpu.VMEM((FB, F), jnp.float32),
        pltpu.VMEM((ACCR, F), jnp.float32),
        pltpu.SemaphoreType.DMA,
        pltpu.SemaphoreType.DMA,
    ],
)


def _t1_body(s1_ref, deg_ref, x_ref, wl_ref, wr_ref, b_ref, o_ref):
    a = s1_ref[...] / jnp.maximum(deg_ref[...], 1.0)
    o = a * wl_ref[...] + x_ref[...] * wr_ref[...] + b_ref[...]
    o_ref[...] = jnp.maximum(o, 0.0)


def _first_conv(s1, deg, x, wl, wr, b):
    R = 1000
    return pl.pallas_call(
        _t1_body,
        grid=(N0 // R,),
        in_specs=[
            pl.BlockSpec((R, 1), lambda g: (g, 0)),
            pl.BlockSpec((R, 1), lambda g: (g, 0)),
            pl.BlockSpec((R, 1), lambda g: (g, 0)),
            pl.BlockSpec((1, F), lambda g: (0, 0)),
            pl.BlockSpec((1, F), lambda g: (0, 0)),
            pl.BlockSpec((1, F), lambda g: (0, 0)),
        ],
        out_specs=pl.BlockSpec((R, F), lambda g: (g, 0)),
        out_shape=jax.ShapeDtypeStruct((N0, F), jnp.float32),
    )(s1, deg, x, wl, wr, b)


def _t2_body(agg_ref, deg_ref, x_ref, wl_ref, wr_ref, b_ref, o_ref):
    a = agg_ref[...] / jnp.maximum(deg_ref[...], 1.0)
    o = (jnp.dot(a, wl_ref[...], preferred_element_type=jnp.float32)
         + jnp.dot(x_ref[...], wr_ref[...], preferred_element_type=jnp.float32)
         + b_ref[...])
    o_ref[...] = jnp.maximum(o, 0.0)


def _sage_mm(agg, deg, x, wl, wr, b):
    R = 1000
    return pl.pallas_call(
        _t2_body,
        grid=(N0 // R,),
        in_specs=[
            pl.BlockSpec((R, F), lambda g: (g, 0)),
            pl.BlockSpec((R, 1), lambda g: (g, 0)),
            pl.BlockSpec((R, F), lambda g: (g, 0)),
            pl.BlockSpec((F, F), lambda g: (0, 0)),
            pl.BlockSpec((F, F), lambda g: (0, 0)),
            pl.BlockSpec((1, F), lambda g: (0, 0)),
        ],
        out_specs=pl.BlockSpec((R, F), lambda g: (g, 0)),
        out_shape=jax.ShapeDtypeStruct((N0, F), jnp.float32),
    )(agg, deg, x, wl, wr, b)


def _t3_body(at_ref, xk_ref, xi_ref, wl_ref, wr_ref, b_ref, o_ref, acc, dacc):
    k = pl.program_id(1)
    nk = pl.num_programs(1)

    @pl.when(k == 0)
    def _():
        acc[...] = jnp.zeros_like(acc)
        dacc[...] = jnp.zeros_like(dacc)

    a = jnp.minimum(at_ref[...], 1.0)
    acc[...] += jnp.dot(a, xk_ref[...], preferred_element_type=jnp.float32)
    dacc[...] += jnp.sum(a, axis=1, keepdims=True)

    @pl.when(k == nk - 1)
    def _():
        m = acc[...] / jnp.maximum(dacc[...], 1.0)
        o = (jnp.dot(m, wl_ref[...], preferred_element_type=jnp.float32)
             + jnp.dot(xi_ref[...], wr_ref[...], preferred_element_type=jnp.float32)
             + b_ref[...])
        o_ref[...] = jnp.maximum(o, 0.0)


def _dense_sage(at, x, wl, wr, b, n, r):
    g = n // r
    return pl.pallas_call(
        _t3_body,
        grid=(g, g),
        in_specs=[
            pl.BlockSpec((r, r), lambda i, k: (i, k)),
            pl.BlockSpec((r, F), lambda i, k: (k, 0)),
            pl.BlockSpec((r, F), lambda i, k: (i, 0)),
            pl.BlockSpec((F, F), lambda i, k: (0, 0)),
            pl.BlockSpec((F, F), lambda i, k: (0, 0)),
            pl.BlockSpec((1, F), lambda i, k: (0, 0)),
        ],
        out_specs=pl.BlockSpec((r, F), lambda i, k: (i, 0)),
        out_shape=jax.ShapeDtypeStruct((n, F), jnp.float32),
        scratch_shapes=[
            pltpu.VMEM((r, F), jnp.float32),
            pltpu.VMEM((r, 1), jnp.float32),
        ],
    )(at, x, x, wl, wr, b)


def _t4_body(x_ref, o_ref):
    o_ref[...] = (x_ref[:, 0, :] + x_ref[:, 1, :]) * 0.5


def _pool(x, r):
    n2 = x.shape[0] // 2
    x3 = x.reshape(n2, 2, F)
    return pl.pallas_call(
        _t4_body,
        grid=(n2 // r,),
        in_specs=[pl.BlockSpec((r, 2, F), lambda g: (g, 0, 0))],
        out_specs=pl.BlockSpec((r, F), lambda g: (g, 0)),
        out_shape=jax.ShapeDtypeStruct((n2, F), jnp.float32),
    )(x3)


def _t6_body(xc_ref, xi_ref, o_ref):
    o_ref[...] = (xc_ref[...][:, None, :] + xi_ref[...]) * 0.5


def _unpool(xc, xinfo, r):
    nc = xc.shape[0]
    xi3 = xinfo.reshape(nc, 2, F)
    out = pl.pallas_call(
        _t6_body,
        grid=(nc // r,),
        in_specs=[
            pl.BlockSpec((r, F), lambda g: (g, 0)),
            pl.BlockSpec((r, 2, F), lambda g: (g, 0, 0)),
        ],
        out_specs=pl.BlockSpec((r, 2, F), lambda g: (g, 0, 0)),
        out_shape=jax.ShapeDtypeStruct((nc, 2, F), jnp.float32),
    )(xc, xi3)
    return out.reshape(nc * 2, F)


def _t8_body(x_ref, w1_ref, b1_ref, w2_ref, b2_ref, w3_ref, b3_ref,
             wf_ref, bf_ref, o_ref):
    h = jnp.maximum(jnp.dot(x_ref[...], w1_ref[...],
                            preferred_element_type=jnp.float32) + b1_ref[...], 0.0)
    h = jnp.maximum(jnp.dot(h, w2_ref[...],
                            preferred_element_type=jnp.float32) + b2_ref[...], 0.0)
    h = jnp.maximum(jnp.dot(h, w3_ref[...],
                            preferred_element_type=jnp.float32) + b3_ref[...], 0.0)
    l = jnp.dot(h, wf_ref[...], preferred_element_type=jnp.float32) + bf_ref[...]
    m = jnp.max(l, axis=1, keepdims=True)
    e = jnp.exp(l - m)
    o_ref[...] = e / jnp.sum(e, axis=1, keepdims=True)


def _mlp(x, w1, b1, w2, b2, w3, b3, wf, bf):
    R = 1000
    return pl.pallas_call(
        _t8_body,
        grid=(N0 // R,),
        in_specs=[
            pl.BlockSpec((R, F), lambda g: (g, 0)),
            pl.BlockSpec((F, 256), lambda g: (0, 0)),
            pl.BlockSpec((1, 256), lambda g: (0, 0)),
            pl.BlockSpec((256, 128), lambda g: (0, 0)),
            pl.BlockSpec((1, 128), lambda g: (0, 0)),
            pl.BlockSpec((128, 64), lambda g: (0, 0)),
            pl.BlockSpec((1, 64), lambda g: (0, 0)),
            pl.BlockSpec((64, 2), lambda g: (0, 0)),
            pl.BlockSpec((1, 2), lambda g: (0, 0)),
        ],
        out_specs=pl.BlockSpec((R, 2), lambda g: (g, 0)),
        out_shape=jax.ShapeDtypeStruct((N0, 2), jnp.float32),
    )(x, w1, b1, w2, b2, w3, b3, wf, bf)


def kernel(x, edge_index, batch, W_first_l, W_first_r, b_first,
           W_pre_l, W_pre_r, b_pre, W_post_l, W_post_r, b_post,
           W_coarse_l, W_coarse_r, b_coarse, W1, b1, W2, b2, W3, b3, Wf, bf):
    src = edge_index[0]
    dst = edge_index[1]
    zrow = jnp.zeros((RBW,), jnp.float32)
    zacc = jnp.zeros((ACCR, F), jnp.float32)

    pk, cnt = _sc_part(src, dst)
    a1f, a2f, a3f = _sc_build(pk, cnt, zrow)
    a1t = a1f.reshape(P1, P1)
    a2t = a2f.reshape(P2, P2)
    a3t = a3f.reshape(P3, P3)

    def conv_agg(h):
        return _sc_conv(h, pk, cnt, zacc)[:N0]

    xfirst = jnp.concatenate(
        [x, jnp.ones((N0, 1), jnp.float32), jnp.zeros((N0, F - 2), jnp.float32)],
        axis=1)
    aggs0 = conv_agg(xfirst)
    s1 = aggs0[:, 0:1]
    deg0 = aggs0[:, 1:2]

    h = _first_conv(s1, deg0, x, W_first_l.reshape(1, F),
                    W_first_r.reshape(1, F), b_first.reshape(1, F))
    for i in range(2):
        aggs = conv_agg(h)
        h = _sage_mm(aggs, deg0, h, W_pre_l[i], W_pre_r[i],
                     b_pre[i].reshape(1, F))
    x0 = h

    h = _pool(h, 1000)
    h = jnp.pad(h, ((0, P1 - H0), (0, 0)))
    for i in range(2):
        h = _dense_sage(a1t, h, W_pre_l[i], W_pre_r[i],
                        b_pre[i].reshape(1, F), P1, 1024)
    x1 = h

    h = _pool(h, 512)
    for i in range(2):
        h = _dense_sage(a2t, h, W_pre_l[i], W_pre_r[i],
                        b_pre[i].reshape(1, F), P2, 512)
    x2f = h

    h = _pool(h, 256)
    h = _dense_sage(a3t, h, W_coarse_l, W_coarse_r,
                    b_coarse.reshape(1, F), P3, 256)

    h = _unpool(h, x2f, 256)
    for i in range(2):
        h = _dense_sage(a2t, h, W_post_l[i], W_post_r[i],
                        b_post[i].reshape(1, F), P2, 512)

    h = _unpool(h, x1, 512)
    for i in range(2):
        h = _dense_sage(a1t, h, W_post_l[i], W_post_r[i],
                        b_post[i].reshape(1, F), P1, 1024)

    h = _unpool(h[:H0], x0, 1000)
    for i in range(2):
        aggs = conv_agg(h)
        h = _sage_mm(aggs, deg0, h, W_post_l[i], W_post_r[i],
                     b_post[i].reshape(1, F))

    return _mlp(h, W1, b1.reshape(1, 256), W2, b2.reshape(1, 128),
                W3, b3.reshape(1, 64), Wf, bf.reshape(1, 2))


# speedup vs baseline: 9.6420x; 2.7577x over previous
import functools

import jax
import jax.numpy as jnp
from jax import lax
from jax.experimental import pallas as pl
from jax.experimental.pallas import tpu as pltpu
from jax.experimental.pallas import tpu_sc as plsc

F = 256
N0 = 10000
E = 160000
NC, NS = 2, 16
H0 = N0 // 2

P1, P2, P3 = 5120, 2560, 1280
A1_SZ, A2_SZ, A3_SZ = P1 * P1, P2 * P2, P3 * P3

CH = 80
EPT = E // NS
NCHUNK = EPT // CH
ZCH = 16384

_mesh = plsc.VectorSubcoreMesh(core_axis_name="c", subcore_axis_name="s")


OWN = 320
NW = NC * NS
NP0 = NW * OWN
ACCR = OWN + 8
CH2 = 4000
NCH2 = E // CH2
FB = 64
STG = 5760
_NLP = pltpu.CompilerParams(needs_layout_passes=False)


def _sc_part_body(src_hbm, dst_hbm, pk_hbm, cnt_hbm,
                  s0, d0, s1, d1, stg, cntv, sem0, sem1):
    c = lax.axis_index("c")
    s = lax.axis_index("s")
    w = c * NS + s
    base = w * OWN

    def fire(i, sv, dv, sem):
        pltpu.async_copy(src_hbm.at[pl.ds(i * CH2, CH2)], sv, sem)
        pltpu.async_copy(dst_hbm.at[pl.ds(i * CH2, CH2)], dv, sem)

    def wait2(sv, sem):
        pltpu.make_async_copy(src_hbm.at[pl.ds(0, CH2)], sv, sem).wait()
        pltpu.make_async_copy(src_hbm.at[pl.ds(0, CH2)], sv, sem).wait()

    def process(srcv, dstv, cur):
        def grp(j, cur):
            sl = pl.ds(j * 16, 16)
            dv = dstv[sl]
            sv = srcv[sl]
            dloc = dv - base
            m = (dloc >= 0) & (dloc < OWN)
            key = jnp.where(m, 0, 1).astype(jnp.int32)
            packed = sv * 512 + jnp.where(m, dloc, OWN)
            _, pv = plsc.sort_key_val(key, packed)
            stg[pl.ds(cur, 16)] = pv
            return cur + jnp.sum(m.astype(jnp.int32))
        return lax.fori_loop(0, CH2 // 16, grp, cur)

    fire(0, s0, d0, sem0)
    fire(1, s1, d1, sem1)

    def pair(fp, cur):
        i0 = 2 * fp
        wait2(s0, sem0)
        cur = process(s0, d0, cur)

        @pl.when(i0 + 2 < NCH2)
        def _f0():
            fire(i0 + 2, s0, d0, sem0)
        wait2(s1, sem1)
        cur = process(s1, d1, cur)

        @pl.when(i0 + 3 < NCH2)
        def _f1():
            fire(i0 + 3, s1, d1, sem1)
        return cur

    cur = lax.fori_loop(0, NCH2 // 2, pair, 0)

    for t in range(FB // 16):
        stg[pl.ds(cur + t * 16, 16)] = jnp.full((16,), OWN, jnp.int32)
    pltpu.sync_copy(stg, pk_hbm.at[pl.ds(w * STG, STG)])
    cntv[pl.ds(0, 16)] = jnp.zeros((16,), jnp.int32) + cur
    pltpu.sync_copy(cntv, cnt_hbm.at[pl.ds(w * 16, 16)])


_sc_part = pl.kernel(
    _sc_part_body,
    out_type=(
        jax.ShapeDtypeStruct((NW * STG,), jnp.int32),
        jax.ShapeDtypeStruct((NW * 16,), jnp.int32),
    ),
    mesh=_mesh,
    compiler_params=_NLP,
    scratch_types=[
        pltpu.VMEM((CH2,), jnp.int32),
        pltpu.VMEM((CH2,), jnp.int32),
        pltpu.VMEM((CH2,), jnp.int32),
        pltpu.VMEM((CH2,), jnp.int32),
        pltpu.VMEM((STG,), jnp.int32),
        pltpu.VMEM((16,), jnp.int32),
        pltpu.SemaphoreType.DMA,
        pltpu.SemaphoreType.DMA,
    ],
)


RBW = 40960
BTRASH = RBW


def _sc_build_body(pk_hbm, cnt_hbm, zr_hbm, a0_hbm, a1_hbm, a2_hbm, a3_hbm,
                   stg, cntv, rowb, sem):
    c = lax.axis_index("c")
    s = lax.axis_index("s")
    w = c * NS + s
    pltpu.sync_copy(pk_hbm.at[pl.ds(w * STG, STG)], stg)
    pltpu.sync_copy(cnt_hbm.at[pl.ds(w * 16, 16)], cntv)
    cur = cntv[pl.ds(0, 16)][0]
    iota16 = lax.iota(jnp.int32, 16)
    ones16 = jnp.ones((16,), jnp.float32)

    def slab(a_hbm, shift, cols, rpc, nchunk):
        cw = rpc * cols
        sw = nchunk * cw

        def chunk(ch, carry):
            pltpu.sync_copy(zr_hbm.at[pl.ds(0, cw)], rowb.at[pl.ds(0, cw)])

            def grp(j, inner):
                pk16 = stg[pl.ds(j * 16, 16)]
                valid = (j * 16 + iota16) < cur
                sv = pk16 >> 9
                dloc = pk16 & 511
                ok = valid
                if shift >= 1:
                    ok = ok & ((sv >> 1) != ((dloc >> 1) + 160 * w))
                if shift >= 2:
                    ok = ok & ((sv >> 2) != ((dloc >> 2) + 80 * w))
                if shift >= 3:
                    ok = ok & ((sv >> 3) != ((dloc >> 3) + 40 * w))
                rl = (dloc >> shift) - ch * rpc
                p = sv >> shift
                m = ok & (rl >= 0) & (rl < rpc)
                idx = jnp.where(m, rl * cols + p, BTRASH)
                plsc.addupdate_scatter(rowb, [idx], ones16)
                return inner

            lax.fori_loop(0, STG // 16, grp, 0)
            pltpu.sync_copy(rowb.at[pl.ds(0, cw)],
                            a_hbm.at[pl.ds(w * sw + ch * cw, cw)])
            return carry

        lax.fori_loop(0, nchunk, chunk, 0)

    slab(a0_hbm, 0, NP0, 4, 80)
    slab(a1_hbm, 1, P1, 8, 20)
    slab(a2_hbm, 2, P2, 16, 5)
    slab(a3_hbm, 3, P3, 20, 2)


_sc_build = pl.kernel(
    _sc_build_body,
    out_type=(
        jax.ShapeDtypeStruct((NP0 * NP0,), jnp.float32),
        jax.ShapeDtypeStruct((A1_SZ,), jnp.float32),
        jax.ShapeDtypeStruct((A2_SZ,), jnp.float32),
        jax.ShapeDtypeStruct((A3_SZ,), jnp.float32),
    ),
    mesh=_mesh,
    compiler_params=_NLP,
    scratch_types=[
        pltpu.VMEM((STG,), jnp.int32),
        pltpu.VMEM((16,), jnp.int32),
        pltpu.VMEM((RBW + 16,), jnp.float32),
        pltpu.SemaphoreType.DMA,
    ],
)


def _sc_conv_body(x_hbm, pk_hbm, cnt_hbm, zacc_hbm, out_hbm,
                  stg, gidx, cntv, rows0, rows1, accv, sem0, sem1):
    c = lax.axis_index("c")
    s = lax.axis_index("s")
    w = c * NS + s
    base = w * OWN

    pltpu.sync_copy(pk_hbm.at[pl.ds(w * STG, STG)], stg)
    pltpu.sync_copy(cnt_hbm.at[pl.ds(w * 16, 16)], cntv)
    pltpu.sync_copy(zacc_hbm, accv)

    cur = cntv[pl.ds(0, 16)][0]
    nf = (cur + FB - 1) // FB

    def unpack(g, _):
        gidx[pl.ds(g * 16, 16)] = stg[pl.ds(g * 16, 16)] >> 9
        return _
    lax.fori_loop(0, STG // 16, unpack, 0)

    def fire(f, buf, sem):
        pltpu.async_copy(x_hbm.at[gidx.at[pl.ds(f * FB, FB)]], buf, sem)

    def waitg(buf, sem):
        pltpu.make_async_copy(x_hbm.at[pl.ds(0, FB)], buf, sem).wait()

    def accum(f, buf):
        def per_g(g, __):
            dloc16 = stg[pl.ds(f * FB + g * 16, 16)] & 511
            evec = g * 16 + lax.iota(jnp.int32, 16)

            def per_k(k, ___):
                for u in range(16):
                    cvec = jnp.zeros((16,), jnp.int32) + (k * 16 + u)
                    vals = plsc.load_gather(buf, [evec, cvec])
                    plsc.addupdate_scatter(accv, [dloc16, cvec], vals)
                return ___
            return lax.fori_loop(0, F // 16, per_k, __)
        lax.fori_loop(0, FB // 16, per_g, 0)

    @pl.when(nf > 0)
    def _():
        fire(0, rows0, sem0)

    @pl.when(nf > 1)
    def _():
        fire(1, rows1, sem1)

    def pair(fp, carry):
        f0 = 2 * fp
        f1 = f0 + 1

        @pl.when(f0 < nf)
        def _b0():
            waitg(rows0, sem0)
            accum(f0, rows0)

            @pl.when(f0 + 2 < nf)
            def _b0f():
                fire(f0 + 2, rows0, sem0)

        @pl.when(f1 < nf)
        def _b1():
            waitg(rows1, sem1)
            accum(f1, rows1)

            @pl.when(f1 + 2 < nf)
            def _b1f():
                fire(f1 + 2, rows1, sem1)
        return carry

    lax.fori_loop(0, (STG // FB + 1) // 2, pair, 0)
    pltpu.sync_copy(accv.at[pl.ds(0, OWN)], out_hbm.at[pl.ds(base, OWN)])


_sc_conv = pl.kernel(
    _sc_conv_body,
    out_type=jax.ShapeDtypeStruct((NP0, F), jnp.float32),
    mesh=_mesh,
    compiler_params=_NLP,
    scratch_types=[
        pltpu.VMEM((STG,), jnp.int32),
        pltpu.VMEM((STG,), jnp.int32),
        pltpu.VMEM((16,), jnp.int32),
        pltpu.VMEM((FB, F), jnp.float32),
        pltpu.VMEM((FB, F), jnp.float32),
        pltpu.VMEM((ACCR, F), jnp.float32),
        pltpu.SemaphoreType.DMA,
        pltpu.SemaphoreType.DMA,
    ],
)


def _t1_body(s1_ref, deg_ref, x_ref, wl_ref, wr_ref, b_ref, o_ref):
    a = s1_ref[...] / jnp.maximum(deg_ref[...], 1.0)
    o = a * wl_ref[...] + x_ref[...] * wr_ref[...] + b_ref[...]
    o_ref[...] = jnp.maximum(o, 0.0)


def _first_conv(s1, deg, x, wl, wr, b):
    R = 1000
    return pl.pallas_call(
        _t1_body,
        grid=(N0 // R,),
        in_specs=[
            pl.BlockSpec((R, 1), lambda g: (g, 0)),
            pl.BlockSpec((R, 1), lambda g: (g, 0)),
            pl.BlockSpec((R, 1), lambda g: (g, 0)),
            pl.BlockSpec((1, F), lambda g: (0, 0)),
            pl.BlockSpec((1, F), lambda g: (0, 0)),
            pl.BlockSpec((1, F), lambda g: (0, 0)),
        ],
        out_specs=pl.BlockSpec((R, F), lambda g: (g, 0)),
        out_shape=jax.ShapeDtypeStruct((N0, F), jnp.float32),
    )(s1, deg, x, wl, wr, b)


def _t2_body(agg_ref, deg_ref, x_ref, wl_ref, wr_ref, b_ref, o_ref):
    a = agg_ref[...] / jnp.maximum(deg_ref[...], 1.0)
    o = (jnp.dot(a, wl_ref[...], preferred_element_type=jnp.float32)
         + jnp.dot(x_ref[...], wr_ref[...], preferred_element_type=jnp.float32)
         + b_ref[...])
    o_ref[...] = jnp.maximum(o, 0.0)


def _sage_mm(agg, deg, x, wl, wr, b):
    R = 1000
    return pl.pallas_call(
        _t2_body,
        grid=(N0 // R,),
        in_specs=[
            pl.BlockSpec((R, F), lambda g: (g, 0)),
            pl.BlockSpec((R, 1), lambda g: (g, 0)),
            pl.BlockSpec((R, F), lambda g: (g, 0)),
            pl.BlockSpec((F, F), lambda g: (0, 0)),
            pl.BlockSpec((F, F), lambda g: (0, 0)),
            pl.BlockSpec((1, F), lambda g: (0, 0)),
        ],
        out_specs=pl.BlockSpec((R, F), lambda g: (g, 0)),
        out_shape=jax.ShapeDtypeStruct((N0, F), jnp.float32),
    )(agg, deg, x, wl, wr, b)


def _t3_body(clip, at_ref, xk_ref, xi_ref, wl_ref, wr_ref, b_ref, o_ref,
             acc, dacc):
    k = pl.program_id(1)
    nk = pl.num_programs(1)

    @pl.when(k == 0)
    def _():
        acc[...] = jnp.zeros_like(acc)
        dacc[...] = jnp.zeros_like(dacc)

    a = at_ref[...]
    if clip:
        a = jnp.minimum(a, 1.0)
    acc[...] += jnp.dot(a, xk_ref[...], preferred_element_type=jnp.float32)
    dacc[...] += jnp.sum(a, axis=1, keepdims=True)

    @pl.when(k == nk - 1)
    def _():
        m = acc[...] / jnp.maximum(dacc[...], 1.0)
        o = (jnp.dot(m, wl_ref[...], preferred_element_type=jnp.float32)
             + jnp.dot(xi_ref[...], wr_ref[...], preferred_element_type=jnp.float32)
             + b_ref[...])
        o_ref[...] = jnp.maximum(o, 0.0)


def _dense_sage(at, x, wl, wr, b, n, r, clip=True):
    g = n // r
    return pl.pallas_call(
        functools.partial(_t3_body, clip),
        grid=(g, g),
        in_specs=[
            pl.BlockSpec((r, r), lambda i, k: (i, k)),
            pl.BlockSpec((r, F), lambda i, k: (k, 0)),
            pl.BlockSpec((r, F), lambda i, k: (i, 0)),
            pl.BlockSpec((F, F), lambda i, k: (0, 0)),
            pl.BlockSpec((F, F), lambda i, k: (0, 0)),
            pl.BlockSpec((1, F), lambda i, k: (0, 0)),
        ],
        out_specs=pl.BlockSpec((r, F), lambda i, k: (i, 0)),
        out_shape=jax.ShapeDtypeStruct((n, F), jnp.float32),
        scratch_shapes=[
            pltpu.VMEM((r, F), jnp.float32),
            pltpu.VMEM((r, 1), jnp.float32),
        ],
    )(at, x, x, wl, wr, b)


def _t4_body(x_ref, o_ref):
    o_ref[...] = (x_ref[:, 0, :] + x_ref[:, 1, :]) * 0.5


def _pool(x, r):
    n2 = x.shape[0] // 2
    x3 = x.reshape(n2, 2, F)
    return pl.pallas_call(
        _t4_body,
        grid=(n2 // r,),
        in_specs=[pl.BlockSpec((r, 2, F), lambda g: (g, 0, 0))],
        out_specs=pl.BlockSpec((r, F), lambda g: (g, 0)),
        out_shape=jax.ShapeDtypeStruct((n2, F), jnp.float32),
    )(x3)


def _t6_body(xc_ref, xi_ref, o_ref):
    o_ref[...] = (xc_ref[...][:, None, :] + xi_ref[...]) * 0.5


def _unpool(xc, xinfo, r):
    nc = xc.shape[0]
    xi3 = xinfo.reshape(nc, 2, F)
    out = pl.pallas_call(
        _t6_body,
        grid=(nc // r,),
        in_specs=[
            pl.BlockSpec((r, F), lambda g: (g, 0)),
            pl.BlockSpec((r, 2, F), lambda g: (g, 0, 0)),
        ],
        out_specs=pl.BlockSpec((r, 2, F), lambda g: (g, 0, 0)),
        out_shape=jax.ShapeDtypeStruct((nc, 2, F), jnp.float32),
    )(xc, xi3)
    return out.reshape(nc * 2, F)


def _t8_body(x_ref, w1_ref, b1_ref, w2_ref, b2_ref, w3_ref, b3_ref,
             wf_ref, bf_ref, o_ref):
    h = jnp.maximum(jnp.dot(x_ref[...], w1_ref[...],
                            preferred_element_type=jnp.float32) + b1_ref[...], 0.0)
    h = jnp.maximum(jnp.dot(h, w2_ref[...],
                            preferred_element_type=jnp.float32) + b2_ref[...], 0.0)
    h = jnp.maximum(jnp.dot(h, w3_ref[...],
                            preferred_element_type=jnp.float32) + b3_ref[...], 0.0)
    l = jnp.dot(h, wf_ref[...], preferred_element_type=jnp.float32) + bf_ref[...]
    m = jnp.max(l, axis=1, keepdims=True)
    e = jnp.exp(l - m)
    o_ref[...] = e / jnp.sum(e, axis=1, keepdims=True)


def _mlp(x, w1, b1, w2, b2, w3, b3, wf, bf):
    R = 1000
    return pl.pallas_call(
        _t8_body,
        grid=(N0 // R,),
        in_specs=[
            pl.BlockSpec((R, F), lambda g: (g, 0)),
            pl.BlockSpec((F, 256), lambda g: (0, 0)),
            pl.BlockSpec((1, 256), lambda g: (0, 0)),
            pl.BlockSpec((256, 128), lambda g: (0, 0)),
            pl.BlockSpec((1, 128), lambda g: (0, 0)),
            pl.BlockSpec((128, 64), lambda g: (0, 0)),
            pl.BlockSpec((1, 64), lambda g: (0, 0)),
            pl.BlockSpec((64, 2), lambda g: (0, 0)),
            pl.BlockSpec((1, 2), lambda g: (0, 0)),
        ],
        out_specs=pl.BlockSpec((R, 2), lambda g: (g, 0)),
        out_shape=jax.ShapeDtypeStruct((N0, 2), jnp.float32),
    )(x, w1, b1, w2, b2, w3, b3, wf, bf)


def kernel(x, edge_index, batch, W_first_l, W_first_r, b_first,
           W_pre_l, W_pre_r, b_pre, W_post_l, W_post_r, b_post,
           W_coarse_l, W_coarse_r, b_coarse, W1, b1, W2, b2, W3, b3, Wf, bf):
    src = edge_index[0]
    dst = edge_index[1]
    zrow = jnp.zeros((RBW,), jnp.float32)

    pk, cnt = _sc_part(src, dst)
    a0f, a1f, a2f, a3f = _sc_build(pk, cnt, zrow)
    a0t = a0f.reshape(NP0, NP0)
    a1t = a1f.reshape(P1, P1)
    a2t = a2f.reshape(P2, P2)
    a3t = a3f.reshape(P3, P3)

    xfirst = jnp.pad(x, ((0, NP0 - N0), (0, F - 1)))
    wl_first = jnp.concatenate([W_first_l, jnp.zeros((F - 1, F), jnp.float32)])
    wr_first = jnp.concatenate([W_first_r, jnp.zeros((F - 1, F), jnp.float32)])
    h = _dense_sage(a0t, xfirst, wl_first, wr_first,
                    b_first.reshape(1, F), NP0, 1024, clip=False)
    for i in range(2):
        h = _dense_sage(a0t, h, W_pre_l[i], W_pre_r[i],
                        b_pre[i].reshape(1, F), NP0, 1024, clip=False)
    x0 = h

    h = _pool(h, 512)
    for i in range(2):
        h = _dense_sage(a1t, h, W_pre_l[i], W_pre_r[i],
                        b_pre[i].reshape(1, F), P1, 1024)
    x1 = h

    h = _pool(h, 512)
    for i in range(2):
        h = _dense_sage(a2t, h, W_pre_l[i], W_pre_r[i],
                        b_pre[i].reshape(1, F), P2, 512)
    x2f = h

    h = _pool(h, 256)
    h = _dense_sage(a3t, h, W_coarse_l, W_coarse_r,
                    b_coarse.reshape(1, F), P3, 256)

    h = _unpool(h, x2f, 256)
    for i in range(2):
        h = _dense_sage(a2t, h, W_post_l[i], W_post_r[i],
                        b_post[i].reshape(1, F), P2, 512)

    h = _unpool(h, x1, 512)
    for i in range(2):
        h = _dense_sage(a1t, h, W_post_l[i], W_post_r[i],
                        b_post[i].reshape(1, F), P1, 1024)

    h = _unpool(h, x0, 512)
    for i in range(2):
        h = _dense_sage(a0t, h, W_post_l[i], W_post_r[i],
                        b_post[i].reshape(1, F), NP0, 1024, clip=False)

    return _mlp(h[:N0], W1, b1.reshape(1, 256), W2, b2.reshape(1, 128),
                W3, b3.reshape(1, 64), Wf, bf.reshape(1, 2))
